# Initial kernel scaffold; baseline (speedup 1.0000x reference)
import functools

import jax
import jax.numpy as jnp
from jax import lax
from jax.experimental import pallas as pl
from jax.experimental.pallas import tpu as pltpu
from jax.experimental.pallas import tpu_sc as plsc

N = 10000
E = 320000
D = 128
G = 64
C = 16

NC = 2
NS = 16
NW = NC * NS

K = 128
CH = 79
EPT = CH * K
EPAD = NW * EPT
NPAD = 10240
RPT = NPAD // NS

ROW_BLK = 500


def _sc_mesh():
    return plsc.VectorSubcoreMesh(core_axis_name="c", subcore_axis_name="s")


@functools.partial(
    pl.kernel,
    mesh=_sc_mesh(),
    out_type=jax.ShapeDtypeStruct((NC, NPAD, 16), jnp.float32),
    scratch_types=[
        pltpu.VMEM((CH, K), jnp.int32),
        pltpu.VMEM((K, 16), jnp.float32),
        pltpu.VMEM_SHARED((NPAD, 16), jnp.float32),
        pltpu.SemaphoreType.DMA,
    ],
)
def _sc_degree(dst_hbm, ones_hbm, zeros_hbm, out_hbm, dst_v, ones_v, acc_sh, sem):
    c = lax.axis_index("c")
    s = lax.axis_index("s")
    wid = s * NC + c
    pltpu.sync_copy(zeros_hbm, acc_sh.at[pl.ds(s * RPT, RPT)])
    pltpu.sync_copy(dst_hbm.at[wid], dst_v)
    pltpu.sync_copy(ones_hbm, ones_v)
    plsc.subcore_barrier()

    def chunk(j, carry):
        pltpu.sync_copy(ones_v, acc_sh.at[dst_v.at[j]], add=True)
        return carry

    lax.fori_loop(0, CH, chunk, 0)
    plsc.subcore_barrier()
    pltpu.sync_copy(acc_sh.at[pl.ds(s * RPT, RPT)],
                    out_hbm.at[c, pl.ds(s * RPT, RPT)])


@functools.partial(
    pl.kernel,
    mesh=_sc_mesh(),
    out_type=jax.ShapeDtypeStruct((NC, NPAD, D), jnp.float32),
    scratch_types=[
        pltpu.VMEM((CH, K), jnp.int32),
        pltpu.VMEM((CH, K), jnp.int32),
        pltpu.VMEM((K, D), jnp.float32),
        pltpu.VMEM_SHARED((NPAD, D), jnp.float32),
        pltpu.SemaphoreType.DMA,
    ],
)
def _sc_scatter(g_hbm, src_hbm, dst_hbm, zeros_hbm, out_hbm,
                src_v, dst_v, rows_v, acc_sh, sem):
    c = lax.axis_index("c")
    s = lax.axis_index("s")
    wid = s * NC + c
    pltpu.sync_copy(zeros_hbm, acc_sh.at[pl.ds(s * RPT, RPT)])
    pltpu.sync_copy(src_hbm.at[wid], src_v)
    pltpu.sync_copy(dst_hbm.at[wid], dst_v)
    plsc.subcore_barrier()

    def chunk(j, carry):
        pltpu.async_copy(g_hbm.at[src_v.at[j]], rows_v, sem).wait()
        pltpu.sync_copy(rows_v, acc_sh.at[dst_v.at[j]], add=True)
        return carry

    lax.fori_loop(0, CH, chunk, 0)
    plsc.subcore_barrier()
    pltpu.sync_copy(acc_sh.at[pl.ds(s * RPT, RPT)],
                    out_hbm.at[c, pl.ds(s * RPT, RPT)])


def _dis_block(deg_a, deg_b):
    deg = deg_a[:, :1] + deg_b[:, :1] + 1.0
    return lax.rsqrt(deg)


def _tc_encode_body(x_ref, we_ref, w0_ref, be_ref, dega_ref, degb_ref, g_ref):
    h = jnp.dot(x_ref[...], we_ref[...], preferred_element_type=jnp.float32)
    h = h + be_ref[...]
    dis = _dis_block(dega_ref[...], degb_ref[...])
    g_ref[...] = dis * jnp.dot(h, w0_ref[...], preferred_element_type=jnp.float32)


def _tc_layer_body(sa_ref, sb_ref, g_ref, dega_ref, degb_ref, b_ref, w_ref,
                   gout_ref):
    dis = _dis_block(dega_ref[...], degb_ref[...])
    h = dis * (sa_ref[...] + sb_ref[...] + g_ref[...]) + b_ref[...]
    h = jnp.maximum(h, 0.0)
    gout_ref[...] = dis * jnp.dot(h, w_ref[...], preferred_element_type=jnp.float32)


def _tc_pool_body(sa_ref, sb_ref, g_ref, dega_ref, degb_ref, b_ref, batch_ref,
                  pool_ref):
    i = pl.program_id(0)
    dis = _dis_block(dega_ref[...], degb_ref[...])
    h = dis * (sa_ref[...] + sb_ref[...] + g_ref[...]) + b_ref[...]
    h = jnp.maximum(h, 0.0)
    gid = lax.broadcasted_iota(jnp.int32, (G, 1), 0)
    onehot = (batch_ref[...] == gid).astype(jnp.float32)
    p = jnp.dot(onehot, h, preferred_element_type=jnp.float32)

    @pl.when(i == 0)
    def _():
        pool_ref[...] = p

    @pl.when(i > 0)
    def _():
        pool_ref[...] += p


def _tc_readout_body(pool_ref, wr1_ref, br1_ref, wr2_ref, br2_ref, out_ref):
    t = jnp.dot(pool_ref[...], wr1_ref[...], preferred_element_type=jnp.float32)
    t = jnp.maximum(t + br1_ref[...], 0.0)
    out_ref[...] = jnp.dot(t, wr2_ref[...],
                           preferred_element_type=jnp.float32) + br2_ref[...]


def _row_spec():
    return pl.BlockSpec((ROW_BLK, D), lambda i: (i, 0))


def _deg_spec():
    return pl.BlockSpec((ROW_BLK, 16), lambda i: (i, 0))


def _full_spec(shape):
    return pl.BlockSpec(shape, lambda i: tuple(0 for _ in shape))


def _tc_encode(x, we, w0, be, dega, degb):
    return pl.pallas_call(
        _tc_encode_body,
        grid=(N // ROW_BLK,),
        in_specs=[_row_spec(), _full_spec((D, D)), _full_spec((D, D)),
                  _full_spec((1, D)), _deg_spec(), _deg_spec()],
        out_specs=_row_spec(),
        out_shape=jax.ShapeDtypeStruct((N, D), jnp.float32),
    )(x, we, w0, be, dega, degb)


def _tc_layer(sa, sb, g, dega, degb, b, w):
    return pl.pallas_call(
        _tc_layer_body,
        grid=(N // ROW_BLK,),
        in_specs=[_row_spec(), _row_spec(), _row_spec(), _deg_spec(),
                  _deg_spec(), _full_spec((1, D)), _full_spec((D, D))],
        out_specs=_row_spec(),
        out_shape=jax.ShapeDtypeStruct((N, D), jnp.float32),
    )(sa, sb, g, dega, degb, b, w)


def _tc_pool(sa, sb, g, dega, degb, b, batch2d):
    return pl.pallas_call(
        _tc_pool_body,
        grid=(N // ROW_BLK,),
        in_specs=[_row_spec(), _row_spec(), _row_spec(), _deg_spec(),
                  _deg_spec(), _full_spec((1, D)),
                  pl.BlockSpec((1, ROW_BLK), lambda i: (0, i))],
        out_specs=_full_spec((G, D)),
        out_shape=jax.ShapeDtypeStruct((G, D), jnp.float32),
    )(sa, sb, g, dega, degb, b, batch2d)


def _tc_readout(pooled, wr1, br1, wr2, br2):
    return pl.pallas_call(
        _tc_readout_body,
        grid=(1,),
        in_specs=[_full_spec((G, D)), _full_spec((D, D)), _full_spec((1, D)),
                  _full_spec((D, C)), _full_spec((1, C))],
        out_specs=_full_spec((G, C)),
        out_shape=jax.ShapeDtypeStruct((G, C), jnp.float32),
    )(pooled, wr1, br1, wr2, br2)


def kernel(x, c_2, u_2, batch, We, be, W0, b0, W1, b1, W2, b2, Wr1, br1, Wr2, br2):
    pad = EPAD - E
    src_t = jnp.concatenate(
        [c_2, jnp.zeros((pad,), jnp.int32)]).reshape(NW, CH, K)
    dst_t = jnp.concatenate(
        [u_2, jnp.full((pad,), N, jnp.int32)]).reshape(NW, CH, K)

    ones16 = jnp.ones((K, 16), jnp.float32)
    zeros16 = jnp.zeros((RPT, 16), jnp.float32)
    zerosD = jnp.zeros((RPT, D), jnp.float32)

    deg_parts = _sc_degree(dst_t, ones16, zeros16)
    dega = deg_parts[0]
    degb = deg_parts[1]

    be2 = be.reshape(1, D)
    b02 = b0.reshape(1, D)
    b12 = b1.reshape(1, D)
    b22 = b2.reshape(1, D)
    br12 = br1.reshape(1, D)
    br22 = br2.reshape(1, C)
    batch2d = batch.reshape(1, N)

    g0 = _tc_encode(x, We, W0, be2, dega, degb)
    s0 = _sc_scatter(g0, src_t, dst_t, zerosD)
    g1 = _tc_layer(s0[0, :N], s0[1, :N], g0, dega, degb, b02, W1)
    s1 = _sc_scatter(g1, src_t, dst_t, zerosD)
    g2 = _tc_layer(s1[0, :N], s1[1, :N], g1, dega, degb, b12, W2)
    s2 = _sc_scatter(g2, src_t, dst_t, zerosD)
    pooled = _tc_pool(s2[0, :N], s2[1, :N], g2, dega, degb, b22, batch2d)
    return _tc_readout(pooled, Wr1, br12, Wr2, br22)


# speedup vs baseline: 9.7202x; 9.7202x over previous
import functools

import jax
import jax.numpy as jnp
from jax import lax
from jax.experimental import pallas as pl
from jax.experimental.pallas import tpu as pltpu
from jax.experimental.pallas import tpu_sc as plsc

N = 10000
E = 320000
D = 128
G = 64
C = 16

NC = 2
NS = 16
NW = NC * NS

K = 128
CH = 79
EPT = CH * K
EPAD = NW * EPT
NPAD = 10240
RPT = NPAD // NS

ROW_BLK = 512


def _sc_mesh():
    return plsc.VectorSubcoreMesh(core_axis_name="c", subcore_axis_name="s")


@functools.partial(
    pl.kernel,
    mesh=_sc_mesh(),
    out_type=jax.ShapeDtypeStruct((NC, NPAD, D), jnp.float32),
    scratch_types=[
        pltpu.VMEM((CH, K), jnp.int32),
        pltpu.VMEM((K, D), jnp.float32),
        pltpu.VMEM_SHARED((NPAD, D), jnp.float32),
        pltpu.SemaphoreType.DMA,
    ],
)
def _sc_degree(dst_hbm, ones_hbm, zeros_hbm, out_hbm, dst_v, ones_v, acc_sh, sem):
    c = lax.axis_index("c")
    s = lax.axis_index("s")
    wid = s * NC + c
    pltpu.sync_copy(zeros_hbm, acc_sh.at[pl.ds(s * RPT, RPT)])
    pltpu.sync_copy(dst_hbm.at[wid], dst_v)
    pltpu.sync_copy(ones_hbm, ones_v)
    plsc.subcore_barrier()

    def chunk(j, carry):
        pltpu.sync_copy(ones_v, acc_sh.at[dst_v.at[j]], add=True)
        return carry

    lax.fori_loop(0, CH, chunk, 0)
    plsc.subcore_barrier()
    pltpu.sync_copy(acc_sh.at[pl.ds(s * RPT, RPT)],
                    out_hbm.at[c, pl.ds(s * RPT, RPT)])


@functools.partial(
    pl.kernel,
    mesh=_sc_mesh(),
    out_type=jax.ShapeDtypeStruct((NC, NPAD, D), jnp.float32),
    scratch_types=[
        pltpu.VMEM((CH, K), jnp.int32),
        pltpu.VMEM((CH, K), jnp.int32),
        pltpu.VMEM((K, D), jnp.float32),
        pltpu.VMEM_SHARED((NPAD, D), jnp.float32),
        pltpu.SemaphoreType.DMA,
    ],
)
def _sc_scatter(g_hbm, src_hbm, dst_hbm, zeros_hbm, out_hbm,
                src_v, dst_v, rows_v, acc_sh, sem):
    c = lax.axis_index("c")
    s = lax.axis_index("s")
    wid = s * NC + c
    pltpu.sync_copy(zeros_hbm, acc_sh.at[pl.ds(s * RPT, RPT)])
    pltpu.sync_copy(src_hbm.at[wid], src_v)
    pltpu.sync_copy(dst_hbm.at[wid], dst_v)
    plsc.subcore_barrier()

    def chunk(j, carry):
        pltpu.async_copy(g_hbm.at[src_v.at[j]], rows_v, sem).wait()
        pltpu.sync_copy(rows_v, acc_sh.at[dst_v.at[j]], add=True)
        return carry

    lax.fori_loop(0, CH, chunk, 0)
    plsc.subcore_barrier()
    pltpu.sync_copy(acc_sh.at[pl.ds(s * RPT, RPT)],
                    out_hbm.at[c, pl.ds(s * RPT, RPT)])


def _dis_block(deg_a, deg_b):
    deg = deg_a[:, :1] + deg_b[:, :1] + 1.0
    return lax.rsqrt(deg)


def _tc_encode_body(x_ref, we_ref, w0_ref, be_ref, dega_ref, degb_ref,
                    g_ref, dis_ref):
    h = jnp.dot(x_ref[...], we_ref[...], preferred_element_type=jnp.float32)
    h = h + be_ref[...]
    dis = _dis_block(dega_ref[...], degb_ref[...])
    dis_ref[...] = jnp.broadcast_to(dis, (ROW_BLK, 16))
    g_ref[...] = dis * jnp.dot(h, w0_ref[...], preferred_element_type=jnp.float32)


def _tc_layer_body(sa_ref, sb_ref, g_ref, dis_in_ref, b_ref, w_ref,
                   gout_ref):
    dis = dis_in_ref[:, :1]
    h = dis * (sa_ref[...] + sb_ref[...] + g_ref[...]) + b_ref[...]
    h = jnp.maximum(h, 0.0)
    gout_ref[...] = dis * jnp.dot(h, w_ref[...], preferred_element_type=jnp.float32)


def _tc_pool_body(sa_ref, sb_ref, g_ref, dis_in_ref, b_ref, batch_ref,
                  pool_ref):
    i = pl.program_id(0)
    dis = dis_in_ref[:, :1]
    h = dis * (sa_ref[...] + sb_ref[...] + g_ref[...]) + b_ref[...]
    h = jnp.maximum(h, 0.0)
    gid = lax.broadcasted_iota(jnp.int32, (G, 1), 0)
    onehot = (batch_ref[...] == gid).astype(jnp.float32)
    p = jnp.dot(onehot, h, preferred_element_type=jnp.float32)

    @pl.when(i == 0)
    def _():
        pool_ref[...] = p

    @pl.when(i > 0)
    def _():
        pool_ref[...] += p


def _tc_readout_body(pool_ref, wr1_ref, br1_ref, wr2_ref, br2_ref, out_ref):
    t = jnp.dot(pool_ref[...], wr1_ref[...], preferred_element_type=jnp.float32)
    t = jnp.maximum(t + br1_ref[...], 0.0)
    out_ref[...] = jnp.dot(t, wr2_ref[...],
                           preferred_element_type=jnp.float32) + br2_ref[...]


def _row_spec():
    return pl.BlockSpec((ROW_BLK, D), lambda i: (i, 0))


def _deg_spec():
    return pl.BlockSpec((ROW_BLK, 16), lambda i: (i, 0))


def _full_spec(shape):
    return pl.BlockSpec(shape, lambda i: tuple(0 for _ in shape))


def _tc_encode(x, we, w0, be, dega, degb):
    return pl.pallas_call(
        _tc_encode_body,
        grid=(NPAD // ROW_BLK,),
        in_specs=[_row_spec(), _full_spec((D, D)), _full_spec((D, D)),
                  _full_spec((1, D)), _row_spec(), _row_spec()],
        out_specs=[_row_spec(), _deg_spec()],
        out_shape=[jax.ShapeDtypeStruct((NPAD, D), jnp.float32),
                   jax.ShapeDtypeStruct((NPAD, 16), jnp.float32)],
    )(x, we, w0, be, dega, degb)


def _tc_layer(sa, sb, g, dis, b, w):
    return pl.pallas_call(
        _tc_layer_body,
        grid=(NPAD // ROW_BLK,),
        in_specs=[_row_spec(), _row_spec(), _row_spec(), _deg_spec(),
                  _full_spec((1, D)), _full_spec((D, D))],
        out_specs=_row_spec(),
        out_shape=jax.ShapeDtypeStruct((NPAD, D), jnp.float32),
    )(sa, sb, g, dis, b, w)


def _tc_pool(sa, sb, g, dis, b, batch2d):
    return pl.pallas_call(
        _tc_pool_body,
        grid=(NPAD // ROW_BLK,),
        in_specs=[_row_spec(), _row_spec(), _row_spec(), _deg_spec(),
                  _full_spec((1, D)),
                  pl.BlockSpec((1, ROW_BLK), lambda i: (0, i))],
        out_specs=_full_spec((G, D)),
        out_shape=jax.ShapeDtypeStruct((G, D), jnp.float32),
    )(sa, sb, g, dis, b, batch2d)


def _tc_readout(pooled, wr1, br1, wr2, br2):
    return pl.pallas_call(
        _tc_readout_body,
        grid=(1,),
        in_specs=[_full_spec((G, D)), _full_spec((D, D)), _full_spec((1, D)),
                  _full_spec((D, C)), _full_spec((1, C))],
        out_specs=_full_spec((G, C)),
        out_shape=jax.ShapeDtypeStruct((G, C), jnp.float32),
    )(pooled, wr1, br1, wr2, br2)


def kernel(x, c_2, u_2, batch, We, be, W0, b0, W1, b1, W2, b2, Wr1, br1, Wr2, br2):
    pad = EPAD - E
    src_t = jnp.concatenate(
        [c_2, jnp.zeros((pad,), jnp.int32)]).reshape(NW, CH, K)
    dst_t = jnp.concatenate(
        [u_2, jnp.full((pad,), N, jnp.int32)]).reshape(NW, CH, K)

    onesD = jnp.ones((K, D), jnp.float32)
    zerosD = jnp.zeros((RPT, D), jnp.float32)

    deg_parts = _sc_degree(dst_t, onesD, zerosD)
    dega = deg_parts[0]
    degb = deg_parts[1]

    be2 = be.reshape(1, D)
    b02 = b0.reshape(1, D)
    b12 = b1.reshape(1, D)
    b22 = b2.reshape(1, D)
    br12 = br1.reshape(1, D)
    br22 = br2.reshape(1, C)
    batch2d = jnp.concatenate(
        [batch, jnp.full((NPAD - N,), G, jnp.int32)]).reshape(1, NPAD)
    xp = jnp.concatenate([x, jnp.zeros((NPAD - N, D), jnp.float32)])

    g0, dis = _tc_encode(xp, We, W0, be2, dega, degb)
    s0 = _sc_scatter(g0, src_t, dst_t, zerosD)
    g1 = _tc_layer(s0[0], s0[1], g0, dis, b02, W1)
    s1 = _sc_scatter(g1, src_t, dst_t, zerosD)
    g2 = _tc_layer(s1[0], s1[1], g1, dis, b12, W2)
    s2 = _sc_scatter(g2, src_t, dst_t, zerosD)
    pooled = _tc_pool(s2[0], s2[1], g2, dis, b22, batch2d)
    return _tc_readout(pooled, Wr1, br12, Wr2, br22)
